# baseline (speedup 1.0000x reference)
import functools

import jax
import jax.numpy as jnp
from jax import lax
from jax.experimental import pallas as pl
from jax.experimental.pallas import tpu as pltpu
from jax.experimental.pallas import tpu_sc as plsc

N = 10000
E = 160000
D = 256
T = 2000
M = 128

N_PAD = 10240
_CHUNK = 128
_NSUB = 16
_EPW = 10240
_GPW = _EPW // 128
_NPAIR = _GPW // 2


def _spmm_body(yL, yR, col_h, row_h, val_h, zeros_h, outL, outR,
               acc, col_b, row_v0, row_v1, val_v0, val_v1, buf0, buf1,
               sem_g0, sem_g1, sem_s):
    c = lax.axis_index("c")
    s = lax.axis_index("s")

    soff = s * (N_PAD // _NSUB)

    pltpu.sync_copy(zeros_h.at[pl.ds(soff, 640)], acc.at[pl.ds(soff, 640)])

    pltpu.sync_copy(col_h.at[s], col_b)
    plsc.subcore_barrier()

    def fire_gather(k, buf, sem):
        @pl.when(c == 0)
        def _():
            pltpu.async_copy(yL.at[col_b.at[k]], buf, sem)

        @pl.when(c == 1)
        def _():
            pltpu.async_copy(yR.at[col_b.at[k]], buf, sem)

    def wait_gather(buf, sem):
        pltpu.make_async_copy(yL.at[col_b.at[0]], buf, sem).wait()

    def load_rv(k, row_v, val_v):
        pltpu.sync_copy(row_h.at[s, k], row_v)
        pltpu.sync_copy(val_h.at[s, k], val_v)

    def scale(buf, val_v):
        def scale_body(kb, carry2):
            val16 = val_v[0, pl.ds(kb * 16, 16)]
            for i in range(16):
                vbc = lax.gather(
                    val16, jnp.full((16, 1), i, jnp.int32),
                    lax.GatherDimensionNumbers(
                        offset_dims=(), collapsed_slice_dims=(0,),
                        start_index_map=(0,)),
                    slice_sizes=(1,),
                    mode=lax.GatherScatterMode.PROMISE_IN_BOUNDS)
                e = kb * 16 + i
                for j in range(8):
                    sl = pl.ds(j * 16, 16)
                    buf[e, sl] = buf[e, sl] * vbc
            return carry2

        lax.fori_loop(0, _CHUNK // 16, scale_body, 0)

    fire_gather(0, buf0, sem_g0)
    load_rv(0, row_v0, val_v0)

    def pair_body(m, carry):
        k0 = 2 * m
        k1 = k0 + 1
        k2 = k0 + 2
        wait_gather(buf0, sem_g0)
        fire_gather(k1, buf1, sem_g1)
        load_rv(k1, row_v1, val_v1)
        scale(buf0, val_v0)
        d0 = pltpu.async_copy(buf0, acc.at[row_v0.at[0]], sem_s, add=True)
        wait_gather(buf1, sem_g1)
        scale(buf1, val_v1)
        d0.wait()

        @pl.when(k2 < _GPW)
        def _():
            fire_gather(k2, buf0, sem_g0)
            load_rv(k2, row_v0, val_v0)

        pltpu.sync_copy(buf1, acc.at[row_v1.at[0]], add=True)
        return carry

    lax.fori_loop(0, _NPAIR, pair_body, 0)
    plsc.subcore_barrier()

    @pl.when(c == 0)
    def _():
        pltpu.sync_copy(acc.at[pl.ds(soff, 640)], outL.at[pl.ds(soff, 640)])

    @pl.when(c == 1)
    def _():
        pltpu.sync_copy(acc.at[pl.ds(soff, 640)], outR.at[pl.ds(soff, 640)])


def _spmm_sc(yL, yR, col, row, val, zeros):
    mesh = plsc.VectorSubcoreMesh(core_axis_name="c", subcore_axis_name="s",
                                  num_cores=2, num_subcores=_NSUB)
    f = pl.kernel(
        _spmm_body,
        out_type=(jax.ShapeDtypeStruct((N_PAD, 128), jnp.float32),
                  jax.ShapeDtypeStruct((N_PAD, 128), jnp.float32)),
        mesh=mesh,
        scratch_types=[
            pltpu.VMEM_SHARED((N_PAD, 128), jnp.float32),
            pltpu.VMEM((_GPW, 128), jnp.int32),
            pltpu.VMEM((1, 128), jnp.int32),
            pltpu.VMEM((1, 128), jnp.int32),
            pltpu.VMEM((1, 128), jnp.float32),
            pltpu.VMEM((1, 128), jnp.float32),
            pltpu.VMEM((_CHUNK, 128), jnp.float32),
            pltpu.VMEM((_CHUNK, 128), jnp.float32),
            pltpu.SemaphoreType.DMA,
            pltpu.SemaphoreType.DMA,
            pltpu.SemaphoreType.DMA,
        ],
    )
    return f(yL, yR, col, row, val, zeros)


_NB = 2000
_NB2 = 1024
_KB = 512


def _mm_body(sL_ref, sR_ref, b_ref, w_ref, oL_ref, oR_ref):
    x = jnp.concatenate([sL_ref[...], sR_ref[...]], axis=1)
    w = w_ref[...]
    h = jax.nn.relu(
        jnp.dot(x, w, preferred_element_type=jnp.float32) + b_ref[...])
    oL_ref[...] = h[:, :128]
    oR_ref[...] = h[:, 128:]


def _mm(sL, sR, b, w):
    return pl.pallas_call(
        _mm_body,
        grid=(N_PAD // _NB2,),
        in_specs=[
            pl.BlockSpec((_NB2, 128), lambda i: (i, 0)),
            pl.BlockSpec((_NB2, 128), lambda i: (i, 0)),
            pl.BlockSpec((1, D), lambda i: (0, 0)),
            pl.BlockSpec((D, D), lambda i: (0, 0)),
        ],
        out_specs=[
            pl.BlockSpec((_NB2, 128), lambda i: (i, 0)),
            pl.BlockSpec((_NB2, 128), lambda i: (i, 0)),
        ],
        out_shape=(jax.ShapeDtypeStruct((N_PAD, 128), jnp.float32),
                   jax.ShapeDtypeStruct((N_PAD, 128), jnp.float32)),
    )(sL, sR, b, w)


def _gatpool1_body(sL_ref, sR_ref, b2_ref, w2_ref, ws_ref, was_ref, wd_ref,
                   wad_ref, gb_ref, ttb_ref, oL_ref, oR_ref, h1_s, hs_s):
    b2 = b2_ref[...]
    w2 = w2_ref[...]
    ws = ws_ref[...]
    HI = lax.Precision.HIGHEST

    nblk = N_PAD // _KB

    def pass_a(k, carry):
        sl = pl.ds(k * _KB, _KB)
        x = jnp.concatenate([sL_ref[sl, :], sR_ref[sl, :]], axis=1)
        h1 = jax.nn.relu(
            jnp.dot(x, w2, preferred_element_type=jnp.float32) + b2)
        h1_s[sl, :] = h1
        hs_s[sl, :] = jnp.dot(h1, ws, preferred_element_type=jnp.float32)
        return carry

    lax.fori_loop(0, nblk, pass_a, 0)

    def pass_b(k, tt_acc):
        sl = pl.ds(k * _KB, _KB)
        ttb_b = ttb_ref[0, sl]
        tio = lax.broadcasted_iota(jnp.int32, (T, _KB), 0)
        sm = (ttb_b[None, :] == tio).astype(jnp.float32)
        return tt_acc + jnp.dot(sm, h1_s[sl, :],
                                preferred_element_type=jnp.float32,
                                precision=HI)

    tt_attr = jax.nn.relu(
        lax.fori_loop(0, nblk, pass_b, jnp.zeros((T, D), jnp.float32)))
    hd = jnp.dot(tt_attr, wd_ref[...], preferred_element_type=jnp.float32)
    ed = lax.dot_general(hd, wad_ref[...], (((1,), (1,)), ((), ())))

    def pass_c(k, carry):
        denom, unnorm = carry
        sl = pl.ds(k * _KB, _KB)
        hs_b = hs_s[sl, :]
        es_row = lax.dot_general(was_ref[...], hs_b,
                                 (((1,), (1,)), ((), ())))
        e0 = ed + es_row
        e = jnp.where(e0 >= 0, e0, 0.01 * e0)
        ttb_b = ttb_ref[0, sl]
        tio = lax.broadcasted_iota(jnp.int32, (T, _KB), 0)
        sm = ttb_b[None, :] == tio
        a = jnp.where(sm, jnp.exp(e), 0.0)
        denom = denom + jnp.sum(a, axis=1, keepdims=True)
        unnorm = unnorm + jnp.dot(a, hs_b,
                                  preferred_element_type=jnp.float32,
                                  precision=HI)
        return denom, unnorm

    denom, unnorm = lax.fori_loop(
        0, nblk, pass_c,
        (jnp.zeros((T, 1), jnp.float32), jnp.zeros((T, D), jnp.float32)))

    z = unnorm / (denom + 1e-16) + gb_ref[...]
    z = jnp.where(z > 0, z, jnp.exp(z) - 1.0)
    z = jax.nn.relu(z)
    oL_ref[...] = z[:, :128]
    oR_ref[...] = z[:, 128:]


def _gatpool1(sL, sR, b2, w2, ws, was, wd, wad, gb, ttb):
    return pl.pallas_call(
        _gatpool1_body,
        in_specs=[
            pl.BlockSpec((N_PAD, 128), lambda: (0, 0)),
            pl.BlockSpec((N_PAD, 128), lambda: (0, 0)),
            pl.BlockSpec((1, D), lambda: (0, 0)),
            pl.BlockSpec((D, D), lambda: (0, 0)),
            pl.BlockSpec((D, D), lambda: (0, 0)),
            pl.BlockSpec((1, D), lambda: (0, 0)),
            pl.BlockSpec((D, D), lambda: (0, 0)),
            pl.BlockSpec((1, D), lambda: (0, 0)),
            pl.BlockSpec((1, D), lambda: (0, 0)),
            pl.BlockSpec((1, N_PAD), lambda: (0, 0)),
        ],
        out_specs=[
            pl.BlockSpec((T, 128), lambda: (0, 0)),
            pl.BlockSpec((T, 128), lambda: (0, 0)),
        ],
        out_shape=(jax.ShapeDtypeStruct((T, 128), jnp.float32),
                   jax.ShapeDtypeStruct((T, 128), jnp.float32)),
        scratch_shapes=[pltpu.VMEM((N_PAD, D), jnp.float32),
                        pltpu.VMEM((N_PAD, D), jnp.float32)],
    )(sL, sR, b2, w2, ws, was, wd, wad, gb, ttb)


def _gatpool2_body(ttL_ref, ttR_ref, ws_ref, was_ref, wd_ref, wad_ref,
                   gb_ref, tgb_ref, p1_ref, pb1_ref, p2_ref, pb2_ref, y_ref):
    HI = lax.Precision.HIGHEST
    tt = jnp.concatenate([ttL_ref[...], ttR_ref[...]], axis=1)
    hs = jnp.dot(tt, ws_ref[...], preferred_element_type=jnp.float32)

    tgb = tgb_ref[0, :]
    mio = lax.broadcasted_iota(jnp.int32, (M, T), 0)
    sm = tgb[None, :] == mio
    smf = sm.astype(jnp.float32)

    mol_attr = jax.nn.relu(
        jnp.dot(smf, tt, preferred_element_type=jnp.float32,
                precision=HI))
    hd = jnp.dot(mol_attr, wd_ref[...], preferred_element_type=jnp.float32)
    ed = lax.dot_general(hd, wad_ref[...], (((1,), (1,)), ((), ())))
    es_row = lax.dot_general(was_ref[...], hs, (((1,), (1,)), ((), ())))
    e0 = ed + es_row
    e = jnp.where(e0 >= 0, e0, 0.01 * e0)
    a = jnp.where(sm, jnp.exp(e), 0.0)
    denom = jnp.sum(a, axis=1, keepdims=True)
    unnorm = jnp.dot(a, hs, preferred_element_type=jnp.float32,
                     precision=HI)
    z = unnorm / (denom + 1e-16) + gb_ref[...]
    z = jnp.where(z > 0, z, jnp.exp(z) - 1.0)
    mol = jax.nn.relu(z)

    p = jax.nn.relu(
        jnp.dot(mol, p1_ref[...], preferred_element_type=jnp.float32)
        + pb1_ref[...])
    y_ref[...] = (jnp.dot(p, p2_ref[...], preferred_element_type=jnp.float32)
                  + pb2_ref[...])


def _gatpool2(ttL, ttR, ws, was, wd, wad, gb, tgb, p1, pb1, p2, pb2):
    return pl.pallas_call(
        _gatpool2_body,
        in_specs=[
            pl.BlockSpec((T, 128), lambda: (0, 0)),
            pl.BlockSpec((T, 128), lambda: (0, 0)),
            pl.BlockSpec((D, D), lambda: (0, 0)),
            pl.BlockSpec((1, D), lambda: (0, 0)),
            pl.BlockSpec((D, D), lambda: (0, 0)),
            pl.BlockSpec((1, D), lambda: (0, 0)),
            pl.BlockSpec((1, D), lambda: (0, 0)),
            pl.BlockSpec((1, T), lambda: (0, 0)),
            pl.BlockSpec((D, 128), lambda: (0, 0)),
            pl.BlockSpec((1, 128), lambda: (0, 0)),
            pl.BlockSpec((128, 1), lambda: (0, 0)),
            pl.BlockSpec((1, 1), lambda: (0, 0)),
        ],
        out_specs=pl.BlockSpec((M, 1), lambda: (0, 0)),
        out_shape=jax.ShapeDtypeStruct((M, 1), jnp.float32),
    )(ttL, ttR, ws, was, wd, wad, gb, tgb, p1, pb1, p2, pb2)


def kernel(node_attr, adj_index, adj_value, tt_node_batch, tt_graph_batch,
           W1, b1, W2, b2,
           g1Ws, g1Wd, g1as, g1ad, g1b,
           g2Ws, g2Wd, g2as, g2ad, g2b,
           P1, pb1, P2, pb2):
    e_pad = _NSUB * _EPW - E
    idx_fill = (jnp.arange(e_pad, dtype=jnp.int32) * 7) % N
    col = jnp.concatenate([adj_index[1], idx_fill]).reshape(_NSUB, _GPW, 128)
    row = jnp.concatenate(
        [adj_index[0], idx_fill]).reshape(_NSUB, _GPW, 1, 128)
    val = jnp.concatenate(
        [adj_value, jnp.zeros((e_pad,), jnp.float32)]).reshape(
            _NSUB, _GPW, 1, 128)
    zeros = jnp.zeros((N_PAD, 128), jnp.float32)
    ttb_pad = jnp.concatenate(
        [tt_node_batch.reshape(1, N),
         jnp.full((1, N_PAD - N), T, jnp.int32)], axis=1)

    b1r = b1.reshape(1, D)
    b2r = b2.reshape(1, D)

    xL = node_attr[:, :128]
    xR = node_attr[:, 128:]
    s1L, s1R = _spmm_sc(xL, xR, col, row, val, zeros)
    hL, hR = _mm(s1L, s1R, b1r, W1)
    s2L, s2R = _spmm_sc(hL, hR, col, row, val, zeros)

    ttL, ttR = _gatpool1(s2L, s2R, b2r, W2, g1Ws, g1as.reshape(1, D),
                         g1Wd, g1ad.reshape(1, D), g1b.reshape(1, D),
                         ttb_pad)
    y = _gatpool2(ttL, ttR, g2Ws, g2as.reshape(1, D),
                  g2Wd, g2ad.reshape(1, D), g2b.reshape(1, D),
                  tt_graph_batch.reshape(1, T),
                  P1, pb1.reshape(1, 128), P2, pb2.reshape(1, 1))
    return y


# speedup vs baseline: 1.3071x; 1.3071x over previous
import functools

import jax
import jax.numpy as jnp
from jax import lax
from jax.experimental import pallas as pl
from jax.experimental.pallas import tpu as pltpu
from jax.experimental.pallas import tpu_sc as plsc

N = 10000
E = 160000
D = 256
T = 2000
M = 128

N_PAD = 10240
_CHUNK = 128
_NSUB = 16
_EPW = 10240
_GPW = _EPW // 128
_NPAIR = _GPW // 2


def _spmm_body(yL, yR, col_h, row_h, val_h, zeros_h, outL, outR,
               acc, col_b, row_v0, row_v1, val_v0, val_v1, buf0, buf1,
               sem_g0, sem_g1, sem_s):
    c = lax.axis_index("c")
    s = lax.axis_index("s")

    soff = s * (N_PAD // _NSUB)

    pltpu.sync_copy(zeros_h.at[pl.ds(soff, 640)], acc.at[pl.ds(soff, 640)])

    pltpu.sync_copy(col_h.at[s], col_b)
    plsc.subcore_barrier()

    def fire_gather(k, buf, sem):
        @pl.when(c == 0)
        def _():
            pltpu.async_copy(yL.at[col_b.at[k]], buf, sem)

        @pl.when(c == 1)
        def _():
            pltpu.async_copy(yR.at[col_b.at[k]], buf, sem)

    def wait_gather(buf, sem):
        pltpu.make_async_copy(yL.at[col_b.at[0]], buf, sem).wait()

    def load_rv(k, row_v, val_v):
        pltpu.sync_copy(row_h.at[s, k], row_v)
        pltpu.sync_copy(val_h.at[s, k], val_v)

    def scale(buf, val_v):
        @plsc.parallel_loop(0, _CHUNK // 16, unroll=2)
        def scale_body(kb):
            val16 = val_v[0, pl.ds(kb * 16, 16)]
            for i in range(16):
                vbc = lax.gather(
                    val16, jnp.full((16, 1), i, jnp.int32),
                    lax.GatherDimensionNumbers(
                        offset_dims=(), collapsed_slice_dims=(0,),
                        start_index_map=(0,)),
                    slice_sizes=(1,),
                    mode=lax.GatherScatterMode.PROMISE_IN_BOUNDS)
                e = kb * 16 + i
                for j in range(8):
                    sl = pl.ds(j * 16, 16)
                    buf[e, sl] = buf[e, sl] * vbc

    fire_gather(0, buf0, sem_g0)
    load_rv(0, row_v0, val_v0)

    def pair_body(m, carry):
        k0 = 2 * m
        k1 = k0 + 1
        k2 = k0 + 2
        wait_gather(buf0, sem_g0)
        fire_gather(k1, buf1, sem_g1)
        load_rv(k1, row_v1, val_v1)
        scale(buf0, val_v0)
        d0 = pltpu.async_copy(buf0, acc.at[row_v0.at[0]], sem_s, add=True)
        wait_gather(buf1, sem_g1)
        scale(buf1, val_v1)
        d0.wait()

        @pl.when(k2 < _GPW)
        def _():
            fire_gather(k2, buf0, sem_g0)
            load_rv(k2, row_v0, val_v0)

        pltpu.sync_copy(buf1, acc.at[row_v1.at[0]], add=True)
        return carry

    lax.fori_loop(0, _NPAIR, pair_body, 0)
    plsc.subcore_barrier()

    @pl.when(c == 0)
    def _():
        pltpu.sync_copy(acc.at[pl.ds(soff, 640)], outL.at[pl.ds(soff, 640)])

    @pl.when(c == 1)
    def _():
        pltpu.sync_copy(acc.at[pl.ds(soff, 640)], outR.at[pl.ds(soff, 640)])


def _spmm_sc(yL, yR, col, row, val, zeros):
    mesh = plsc.VectorSubcoreMesh(core_axis_name="c", subcore_axis_name="s",
                                  num_cores=2, num_subcores=_NSUB)
    f = pl.kernel(
        _spmm_body,
        out_type=(jax.ShapeDtypeStruct((N_PAD, 128), jnp.float32),
                  jax.ShapeDtypeStruct((N_PAD, 128), jnp.float32)),
        mesh=mesh,
        scratch_types=[
            pltpu.VMEM_SHARED((N_PAD, 128), jnp.float32),
            pltpu.VMEM((_GPW, 128), jnp.int32),
            pltpu.VMEM((1, 128), jnp.int32),
            pltpu.VMEM((1, 128), jnp.int32),
            pltpu.VMEM((1, 128), jnp.float32),
            pltpu.VMEM((1, 128), jnp.float32),
            pltpu.VMEM((_CHUNK, 128), jnp.float32),
            pltpu.VMEM((_CHUNK, 128), jnp.float32),
            pltpu.SemaphoreType.DMA,
            pltpu.SemaphoreType.DMA,
            pltpu.SemaphoreType.DMA,
        ],
    )
    return f(yL, yR, col, row, val, zeros)


_NB = 2000
_NB2 = 1024
_KB = 512


def _mm_body(sL_ref, sR_ref, b_ref, w_ref, oL_ref, oR_ref):
    x = jnp.concatenate([sL_ref[...], sR_ref[...]], axis=1)
    w = w_ref[...]
    h = jax.nn.relu(
        jnp.dot(x, w, preferred_element_type=jnp.float32) + b_ref[...])
    oL_ref[...] = h[:, :128]
    oR_ref[...] = h[:, 128:]


def _mm(sL, sR, b, w):
    return pl.pallas_call(
        _mm_body,
        grid=(N_PAD // _NB2,),
        in_specs=[
            pl.BlockSpec((_NB2, 128), lambda i: (i, 0)),
            pl.BlockSpec((_NB2, 128), lambda i: (i, 0)),
            pl.BlockSpec((1, D), lambda i: (0, 0)),
            pl.BlockSpec((D, D), lambda i: (0, 0)),
        ],
        out_specs=[
            pl.BlockSpec((_NB2, 128), lambda i: (i, 0)),
            pl.BlockSpec((_NB2, 128), lambda i: (i, 0)),
        ],
        out_shape=(jax.ShapeDtypeStruct((N_PAD, 128), jnp.float32),
                   jax.ShapeDtypeStruct((N_PAD, 128), jnp.float32)),
    )(sL, sR, b, w)


def _gatpool1_body(sL_ref, sR_ref, b2_ref, w2_ref, ws_ref, was_ref, wd_ref,
                   wad_ref, gb_ref, ttb_ref, oL_ref, oR_ref, h1_s, hs_s):
    b2 = b2_ref[...]
    w2 = w2_ref[...]
    ws = ws_ref[...]
    HI = lax.Precision.HIGHEST

    nblk = N_PAD // _KB

    def pass_a(k, carry):
        sl = pl.ds(k * _KB, _KB)
        x = jnp.concatenate([sL_ref[sl, :], sR_ref[sl, :]], axis=1)
        h1 = jax.nn.relu(
            jnp.dot(x, w2, preferred_element_type=jnp.float32) + b2)
        h1_s[sl, :] = h1
        hs_s[sl, :] = jnp.dot(h1, ws, preferred_element_type=jnp.float32)
        return carry

    lax.fori_loop(0, nblk, pass_a, 0)

    def pass_b(k, tt_acc):
        sl = pl.ds(k * _KB, _KB)
        ttb_b = ttb_ref[0, sl]
        tio = lax.broadcasted_iota(jnp.int32, (T, _KB), 0)
        sm = (ttb_b[None, :] == tio).astype(jnp.float32)
        h1_b = h1_s[sl, :]
        h1_hi = h1_b.astype(jnp.bfloat16).astype(jnp.float32)
        acc = jnp.dot(sm, h1_hi, preferred_element_type=jnp.float32)
        acc = acc + jnp.dot(sm, h1_b - h1_hi,
                            preferred_element_type=jnp.float32)
        return tt_acc + acc

    tt_attr = jax.nn.relu(
        lax.fori_loop(0, nblk, pass_b, jnp.zeros((T, D), jnp.float32)))
    hd = jnp.dot(tt_attr, wd_ref[...], preferred_element_type=jnp.float32)
    ed = lax.dot_general(hd, wad_ref[...], (((1,), (1,)), ((), ())))

    def pass_c(k, carry):
        denom, unnorm = carry
        sl = pl.ds(k * _KB, _KB)
        hs_b = hs_s[sl, :]
        es_row = lax.dot_general(was_ref[...], hs_b,
                                 (((1,), (1,)), ((), ())))
        e0 = ed + es_row
        e = jnp.where(e0 >= 0, e0, 0.01 * e0)
        ttb_b = ttb_ref[0, sl]
        tio = lax.broadcasted_iota(jnp.int32, (T, _KB), 0)
        sm = ttb_b[None, :] == tio
        a = jnp.where(sm, jnp.exp(e), 0.0)
        denom = denom + jnp.sum(a, axis=1, keepdims=True)
        a_hi = a.astype(jnp.bfloat16).astype(jnp.float32)
        a_lo = a - a_hi
        hs_hi = hs_b.astype(jnp.bfloat16).astype(jnp.float32)
        hs_lo = hs_b - hs_hi
        u = (jnp.dot(a_hi, hs_hi, preferred_element_type=jnp.float32)
             + jnp.dot(a_hi, hs_lo, preferred_element_type=jnp.float32)
             + jnp.dot(a_lo, hs_hi, preferred_element_type=jnp.float32))
        unnorm = unnorm + u
        return denom, unnorm

    denom, unnorm = lax.fori_loop(
        0, nblk, pass_c,
        (jnp.zeros((T, 1), jnp.float32), jnp.zeros((T, D), jnp.float32)))

    z = unnorm / (denom + 1e-16) + gb_ref[...]
    z = jnp.where(z > 0, z, jnp.exp(z) - 1.0)
    z = jax.nn.relu(z)
    oL_ref[...] = z[:, :128]
    oR_ref[...] = z[:, 128:]


def _gatpool1(sL, sR, b2, w2, ws, was, wd, wad, gb, ttb):
    return pl.pallas_call(
        _gatpool1_body,
        in_specs=[
            pl.BlockSpec((N_PAD, 128), lambda: (0, 0)),
            pl.BlockSpec((N_PAD, 128), lambda: (0, 0)),
            pl.BlockSpec((1, D), lambda: (0, 0)),
            pl.BlockSpec((D, D), lambda: (0, 0)),
            pl.BlockSpec((D, D), lambda: (0, 0)),
            pl.BlockSpec((1, D), lambda: (0, 0)),
            pl.BlockSpec((D, D), lambda: (0, 0)),
            pl.BlockSpec((1, D), lambda: (0, 0)),
            pl.BlockSpec((1, D), lambda: (0, 0)),
            pl.BlockSpec((1, N_PAD), lambda: (0, 0)),
        ],
        out_specs=[
            pl.BlockSpec((T, 128), lambda: (0, 0)),
            pl.BlockSpec((T, 128), lambda: (0, 0)),
        ],
        out_shape=(jax.ShapeDtypeStruct((T, 128), jnp.float32),
                   jax.ShapeDtypeStruct((T, 128), jnp.float32)),
        scratch_shapes=[pltpu.VMEM((N_PAD, D), jnp.float32),
                        pltpu.VMEM((N_PAD, D), jnp.float32)],
    )(sL, sR, b2, w2, ws, was, wd, wad, gb, ttb)


def _gatpool2_body(ttL_ref, ttR_ref, ws_ref, was_ref, wd_ref, wad_ref,
                   gb_ref, tgb_ref, p1_ref, pb1_ref, p2_ref, pb2_ref, y_ref):
    HI = lax.Precision.HIGHEST
    tt = jnp.concatenate([ttL_ref[...], ttR_ref[...]], axis=1)
    hs = jnp.dot(tt, ws_ref[...], preferred_element_type=jnp.float32)

    tgb = tgb_ref[0, :]
    mio = lax.broadcasted_iota(jnp.int32, (M, T), 0)
    sm = tgb[None, :] == mio
    smf = sm.astype(jnp.float32)

    mol_attr = jax.nn.relu(
        jnp.dot(smf, tt, preferred_element_type=jnp.float32,
                precision=HI))
    hd = jnp.dot(mol_attr, wd_ref[...], preferred_element_type=jnp.float32)
    ed = lax.dot_general(hd, wad_ref[...], (((1,), (1,)), ((), ())))
    es_row = lax.dot_general(was_ref[...], hs, (((1,), (1,)), ((), ())))
    e0 = ed + es_row
    e = jnp.where(e0 >= 0, e0, 0.01 * e0)
    a = jnp.where(sm, jnp.exp(e), 0.0)
    denom = jnp.sum(a, axis=1, keepdims=True)
    unnorm = jnp.dot(a, hs, preferred_element_type=jnp.float32,
                     precision=HI)
    z = unnorm / (denom + 1e-16) + gb_ref[...]
    z = jnp.where(z > 0, z, jnp.exp(z) - 1.0)
    mol = jax.nn.relu(z)

    p = jax.nn.relu(
        jnp.dot(mol, p1_ref[...], preferred_element_type=jnp.float32)
        + pb1_ref[...])
    y_ref[...] = (jnp.dot(p, p2_ref[...], preferred_element_type=jnp.float32)
                  + pb2_ref[...])


def _gatpool2(ttL, ttR, ws, was, wd, wad, gb, tgb, p1, pb1, p2, pb2):
    return pl.pallas_call(
        _gatpool2_body,
        in_specs=[
            pl.BlockSpec((T, 128), lambda: (0, 0)),
            pl.BlockSpec((T, 128), lambda: (0, 0)),
            pl.BlockSpec((D, D), lambda: (0, 0)),
            pl.BlockSpec((1, D), lambda: (0, 0)),
            pl.BlockSpec((D, D), lambda: (0, 0)),
            pl.BlockSpec((1, D), lambda: (0, 0)),
            pl.BlockSpec((1, D), lambda: (0, 0)),
            pl.BlockSpec((1, T), lambda: (0, 0)),
            pl.BlockSpec((D, 128), lambda: (0, 0)),
            pl.BlockSpec((1, 128), lambda: (0, 0)),
            pl.BlockSpec((128, 1), lambda: (0, 0)),
            pl.BlockSpec((1, 1), lambda: (0, 0)),
        ],
        out_specs=pl.BlockSpec((M, 1), lambda: (0, 0)),
        out_shape=jax.ShapeDtypeStruct((M, 1), jnp.float32),
    )(ttL, ttR, ws, was, wd, wad, gb, tgb, p1, pb1, p2, pb2)


def kernel(node_attr, adj_index, adj_value, tt_node_batch, tt_graph_batch,
           W1, b1, W2, b2,
           g1Ws, g1Wd, g1as, g1ad, g1b,
           g2Ws, g2Wd, g2as, g2ad, g2b,
           P1, pb1, P2, pb2):
    e_pad = _NSUB * _EPW - E
    idx_fill = (jnp.arange(e_pad, dtype=jnp.int32) * 7) % N
    col = jnp.concatenate([adj_index[1], idx_fill]).reshape(_NSUB, _GPW, 128)
    row = jnp.concatenate(
        [adj_index[0], idx_fill]).reshape(_NSUB, _GPW, 1, 128)
    val = jnp.concatenate(
        [adj_value, jnp.zeros((e_pad,), jnp.float32)]).reshape(
            _NSUB, _GPW, 1, 128)
    zeros = jnp.zeros((N_PAD, 128), jnp.float32)
    ttb_pad = jnp.concatenate(
        [tt_node_batch.reshape(1, N),
         jnp.full((1, N_PAD - N), T, jnp.int32)], axis=1)

    b1r = b1.reshape(1, D)
    b2r = b2.reshape(1, D)

    xL = node_attr[:, :128]
    xR = node_attr[:, 128:]
    s1L, s1R = _spmm_sc(xL, xR, col, row, val, zeros)
    hL, hR = _mm(s1L, s1R, b1r, W1)
    s2L, s2R = _spmm_sc(hL, hR, col, row, val, zeros)

    ttL, ttR = _gatpool1(s2L, s2R, b2r, W2, g1Ws, g1as.reshape(1, D),
                         g1Wd, g1ad.reshape(1, D), g1b.reshape(1, D),
                         ttb_pad)
    y = _gatpool2(ttL, ttR, g2Ws, g2as.reshape(1, D),
                  g2Wd, g2ad.reshape(1, D), g2b.reshape(1, D),
                  tt_graph_batch.reshape(1, T),
                  P1, pb1.reshape(1, 128), P2, pb2.reshape(1, 1))
    return y
